# baseline (speedup 1.0000x reference)
import functools

import jax
import jax.numpy as jnp
from jax.experimental import pallas as pl
from jax.experimental.pallas import tpu as pltpu


def _edge_mlp_body(nb, ea, th, w1a, w1b, b1, w2, b2, awm, ab, m_ref, lg_ref):
    f32 = jnp.float32
    bf = jnp.bfloat16
    pre = (
        jax.lax.dot(nb[...].astype(bf), w1a[...].astype(bf),
                    preferred_element_type=f32)
        + jax.lax.dot(ea[...].astype(bf), w1b[...].astype(bf),
                      preferred_element_type=f32)
        + b1[...]
    )
    L = jnp.where(pre >= 0, pre, 0.2 * pre)
    m = jax.lax.dot(L.astype(bf), w2[...].astype(bf),
                    preferred_element_type=f32) + b2[...]
    lg = jax.lax.dot(m.astype(bf), awm[...].astype(bf),
                     preferred_element_type=f32) + th[...] + ab[...]
    m_ref[...] = m
    lg_ref[...] = lg


def _edge_mlp(nb, ea, th, w1a, w1b, b1, w2, b2, awm, ab, block=2000):
    E, H = nb.shape
    HH = w2.shape[1]
    HEADS = awm.shape[1]
    grid = (E // block,)
    row = lambda i: (i, 0)
    fixed = lambda i: (0, 0)
    return pl.pallas_call(
        _edge_mlp_body,
        grid=grid,
        in_specs=[
            pl.BlockSpec((block, H), row),
            pl.BlockSpec((block, ea.shape[1]), row),
            pl.BlockSpec((block, HEADS), row),
            pl.BlockSpec(w1a.shape, fixed),
            pl.BlockSpec(w1b.shape, fixed),
            pl.BlockSpec(b1.shape, fixed),
            pl.BlockSpec(w2.shape, fixed),
            pl.BlockSpec(b2.shape, fixed),
            pl.BlockSpec(awm.shape, fixed),
            pl.BlockSpec(ab.shape, fixed),
        ],
        out_specs=[
            pl.BlockSpec((block, HH), row),
            pl.BlockSpec((block, HEADS), row),
        ],
        out_shape=[
            jax.ShapeDtypeStruct((E, HH), jnp.float32),
            jax.ShapeDtypeStruct((E, HEADS), jnp.float32),
        ],
    )(nb, ea, th, w1a, w1b, b1, w2, b2, awm, ab)


def kernel(x, edge_index, edge_attr, params):
    src, tgt = edge_index[0], edge_index[1]
    H = params['inW'].shape[1]
    E = edge_attr.shape[0]
    h = x @ params['inW'] + params['inb']
    for lp in params['layers']:
        HEADS = lp['aW'].shape[1]
        HH = lp['mW2'].shape[1]
        nb = jnp.take(h, src, axis=0)
        th_tgt = jnp.take(h @ lp['aW'][HH:], tgt, axis=0)
        m, logits = _edge_mlp(
            nb, edge_attr, th_tgt,
            lp['mW1'][:H], lp['mW1'][H:], lp['mb1'].reshape(1, -1),
            lp['mW2'], lp['mb2'].reshape(1, -1),
            lp['aW'][:HH], lp['ab'].reshape(1, -1),
        )
        a = jax.nn.softmax(logits, axis=0)
        mh = m.reshape(E, HEADS, H)
        w = (mh * a[:, :, None]).mean(axis=1)
        agg = jnp.zeros((h.shape[0], H), dtype=h.dtype).at[tgt].add(w)
        u = jnp.concatenate([h, agg], axis=-1) @ lp['uW'] + lp['ub']
        mu = u.mean(axis=-1, keepdims=True)
        var = u.var(axis=-1, keepdims=True)
        u = (u - mu) / jnp.sqrt(var + 1e-5) * lp['ln_g'] + lp['ln_b']
        h = h + jnp.where(u >= 0, u, 0.2 * u)
    t = jax.nn.relu(h @ params['tW1'] + params['tb1']) @ params['tW2'] + params['tb2']
    g_emb = h.mean(axis=0, keepdims=True)
    g = jax.nn.relu(g_emb @ params['gW1'] + params['gb1']) @ params['gW2'] + params['gb2']
    return t.squeeze(-1), h, g.squeeze(0)
